# baseline (speedup 1.0000x reference)
import functools

import jax
import jax.numpy as jnp
from jax import lax
from jax.experimental import pallas as pl
from jax.experimental.pallas import tpu as pltpu
from jax.experimental.pallas import tpu_sc as plsc

HIDDEN = 4096
NUM_CORES = 2
NUM_SUBCORES = 16
NUM_WORKERS = NUM_CORES * NUM_SUBCORES
CHUNK = 8


def _build(num_rows):
    rows_per_w = num_rows // NUM_WORKERS
    nchunk = rows_per_w // CHUNK
    mesh = plsc.VectorSubcoreMesh(core_axis_name="c", subcore_axis_name="s")

    @functools.partial(
        pl.kernel,
        mesh=mesh,
        out_type=jax.ShapeDtypeStruct((num_rows, HIDDEN), jnp.float32),
        scratch_types=[
            pltpu.VMEM((rows_per_w,), jnp.int32),
            pltpu.VMEM((CHUNK, HIDDEN), jnp.float32),
            pltpu.SemaphoreType.DMA,
        ],
    )
    def gather_kernel(idx_hbm, table_hbm, out_hbm, idx_v, buf, gsem):
        wid = lax.axis_index("s") * NUM_CORES + lax.axis_index("c")
        base = wid * rows_per_w
        pltpu.sync_copy(idx_hbm.at[pl.ds(base, rows_per_w)], idx_v)

        def body(c, carry):
            pltpu.async_copy(
                table_hbm.at[idx_v.at[pl.ds(c * CHUNK, CHUNK)]], buf, gsem
            ).wait()
            pltpu.sync_copy(buf, out_hbm.at[pl.ds(base + c * CHUNK, CHUNK)])
            return carry

        lax.fori_loop(0, nchunk, body, 0)

    return gather_kernel


def kernel(input_ids, word_embeddings):
    b, s = input_ids.shape
    perm_idx = input_ids.T.reshape(-1).astype(jnp.int32)
    out = _build(b * s)(perm_idx, word_embeddings)
    return out.reshape(s, b, HIDDEN)


# speedup vs baseline: 1.0987x; 1.0987x over previous
import functools

import jax
import jax.numpy as jnp
from jax import lax
from jax.experimental import pallas as pl
from jax.experimental.pallas import tpu as pltpu
from jax.experimental.pallas import tpu_sc as plsc

HIDDEN = 4096
NUM_CORES = 2
NUM_SUBCORES = 16
NUM_WORKERS = NUM_CORES * NUM_SUBCORES
CHUNK = 8


def _build(num_rows):
    rows_per_w = num_rows // NUM_WORKERS
    nchunk = rows_per_w // CHUNK
    mesh = plsc.VectorSubcoreMesh(core_axis_name="c", subcore_axis_name="s")

    @functools.partial(
        pl.kernel,
        mesh=mesh,
        out_type=jax.ShapeDtypeStruct((num_rows, HIDDEN), jnp.float32),
        scratch_types=[
            pltpu.VMEM((rows_per_w,), jnp.int32),
            pltpu.VMEM((2, CHUNK, HIDDEN), jnp.float32),
            pltpu.SemaphoreType.DMA,
            pltpu.SemaphoreType.DMA,
        ],
    )
    def gather_kernel(idx_hbm, table_hbm, out_hbm, idx_v, bufs, gsem, wsem):
        wid = lax.axis_index("s") * NUM_CORES + lax.axis_index("c")
        base = wid * rows_per_w
        pltpu.sync_copy(idx_hbm.at[pl.ds(base, rows_per_w)], idx_v)

        def start_gather(c, b):
            pltpu.async_copy(
                table_hbm.at[idx_v.at[pl.ds(c * CHUNK, CHUNK)]], bufs.at[b], gsem
            )

        def wait_gather(c, b):
            pltpu.make_async_copy(
                table_hbm.at[idx_v.at[pl.ds(c * CHUNK, CHUNK)]], bufs.at[b], gsem
            ).wait()

        def start_write(c, b):
            pltpu.async_copy(
                bufs.at[b], out_hbm.at[pl.ds(base + c * CHUNK, CHUNK)], wsem
            )

        def wait_write(c, b):
            pltpu.make_async_copy(
                bufs.at[b], out_hbm.at[pl.ds(base + c * CHUNK, CHUNK)], wsem
            ).wait()

        start_gather(0, 0)
        start_gather(1, 1)

        def body(i, carry):
            c0 = i * 2
            for b in range(2):
                c = c0 + b
                wait_gather(c, b)
                start_write(c, b)
                wait_write(c, b)
                start_gather(c + 2, b)
            return carry

        lax.fori_loop(0, (nchunk - 2) // 2, body, 0)

        for b, c in ((0, nchunk - 2), (1, nchunk - 1)):
            wait_gather(c, b)
            start_write(c, b)
        for b, c in ((0, nchunk - 2), (1, nchunk - 1)):
            wait_write(c, b)

    return gather_kernel


def kernel(input_ids, word_embeddings):
    b, s = input_ids.shape
    perm_idx = input_ids.T.reshape(-1).astype(jnp.int32)
    out = _build(b * s)(perm_idx, word_embeddings)
    return out.reshape(s, b, HIDDEN)


# speedup vs baseline: 2.4577x; 2.2369x over previous
import functools

import jax
import jax.numpy as jnp
from jax import lax
from jax.experimental import pallas as pl
from jax.experimental.pallas import tpu as pltpu
from jax.experimental.pallas import tpu_sc as plsc

HIDDEN = 4096
NUM_CORES = 2
NUM_SUBCORES = 16
NUM_WORKERS = NUM_CORES * NUM_SUBCORES
CHUNK = 8


def _build(num_rows):
    rows_per_w = num_rows // NUM_WORKERS
    nchunk = rows_per_w // CHUNK
    mesh = plsc.VectorSubcoreMesh(core_axis_name="c", subcore_axis_name="s")

    @functools.partial(
        pl.kernel,
        mesh=mesh,
        out_type=jax.ShapeDtypeStruct(
            (num_rows // 4, 4, HIDDEN), jnp.float32
        ),
        scratch_types=[
            pltpu.VMEM((rows_per_w,), jnp.int32),
            pltpu.VMEM((2, CHUNK, HIDDEN), jnp.float32),
            pltpu.SemaphoreType.DMA,
            pltpu.SemaphoreType.DMA,
        ],
    )
    def gather_kernel(idx_hbm, table_hbm, out3_hbm, idx_v, bufs, gsem, wsem):
        out_hbm = out3_hbm.reshape(num_rows, HIDDEN)
        wid = lax.axis_index("s") * NUM_CORES + lax.axis_index("c")
        base = wid * rows_per_w
        pltpu.sync_copy(idx_hbm.at[pl.ds(base, rows_per_w)], idx_v)

        def start_gather(c, b):
            pltpu.async_copy(
                table_hbm.at[idx_v.at[pl.ds(c * CHUNK, CHUNK)]], bufs.at[b], gsem
            )

        def wait_gather(c, b):
            pltpu.make_async_copy(
                table_hbm.at[idx_v.at[pl.ds(c * CHUNK, CHUNK)]], bufs.at[b], gsem
            ).wait()

        def start_write(c, b):
            pltpu.async_copy(
                bufs.at[b], out_hbm.at[pl.ds(base + c * CHUNK, CHUNK)], wsem
            )

        def wait_write(c, b):
            pltpu.make_async_copy(
                bufs.at[b], out_hbm.at[pl.ds(base + c * CHUNK, CHUNK)], wsem
            ).wait()

        start_gather(0, 0)
        start_gather(1, 1)

        def body(i, carry):
            c0 = i * 2
            for b in range(2):
                c = c0 + b
                wait_gather(c, b)
                start_write(c, b)
                wait_write(c, b)
                start_gather(c + 2, b)
            return carry

        lax.fori_loop(0, (nchunk - 2) // 2, body, 0)

        for b, c in ((0, nchunk - 2), (1, nchunk - 1)):
            wait_gather(c, b)
            start_write(c, b)
        for b, c in ((0, nchunk - 2), (1, nchunk - 1)):
            wait_write(c, b)

    return gather_kernel


def kernel(input_ids, word_embeddings):
    b, s = input_ids.shape
    perm_idx = input_ids.T.reshape(-1).astype(jnp.int32)
    return _build(b * s)(perm_idx, word_embeddings)


# speedup vs baseline: 2.4746x; 1.0069x over previous
import functools

import jax
import jax.numpy as jnp
from jax import lax
from jax.experimental import pallas as pl
from jax.experimental.pallas import tpu as pltpu
from jax.experimental.pallas import tpu_sc as plsc

HIDDEN = 4096
NUM_CORES = 2
NUM_SUBCORES = 16
NUM_WORKERS = NUM_CORES * NUM_SUBCORES
CHUNK = 8


def _build(num_rows):
    rows_per_w = num_rows // NUM_WORKERS
    nchunk = rows_per_w // CHUNK
    mesh = plsc.VectorSubcoreMesh(core_axis_name="c", subcore_axis_name="s")

    @functools.partial(
        pl.kernel,
        mesh=mesh,
        out_type=jax.ShapeDtypeStruct(
            (num_rows // 4, 4, HIDDEN), jnp.float32
        ),
        scratch_types=[
            pltpu.VMEM((rows_per_w,), jnp.int32),
            pltpu.VMEM((3, CHUNK, HIDDEN), jnp.float32),
            pltpu.SemaphoreType.DMA,
            pltpu.SemaphoreType.DMA,
        ],
    )
    def gather_kernel(idx_hbm, table_hbm, out3_hbm, idx_v, bufs, gsem, wsem):
        out_hbm = out3_hbm.reshape(num_rows, HIDDEN)
        wid = lax.axis_index("s") * NUM_CORES + lax.axis_index("c")
        base = wid * rows_per_w
        pltpu.sync_copy(idx_hbm.at[pl.ds(base, rows_per_w)], idx_v)

        def start_gather(c, b):
            pltpu.async_copy(
                table_hbm.at[idx_v.at[pl.ds(c * CHUNK, CHUNK)]], bufs.at[b], gsem
            )

        def wait_gather(c, b):
            pltpu.make_async_copy(
                table_hbm.at[idx_v.at[pl.ds(c * CHUNK, CHUNK)]], bufs.at[b], gsem
            ).wait()

        def start_write(c, b):
            pltpu.async_copy(
                bufs.at[b], out_hbm.at[pl.ds(base + c * CHUNK, CHUNK)], wsem
            )

        def wait_write(c, b):
            pltpu.make_async_copy(
                bufs.at[b], out_hbm.at[pl.ds(base + c * CHUNK, CHUNK)], wsem
            ).wait()

        start_gather(0, 0)
        start_gather(1, 1)

        wait_gather(0, 0)
        start_write(0, 0)
        start_gather(2, 2)

        def body(i, carry):
            c0 = 1 + i * 3
            for b in range(3):
                c = c0 + b
                slot = (1 + b) % 3
                wait_gather(c, slot)
                start_write(c, slot)
                wait_write(c - 1, b)
                start_gather(c + 2, b)
            return carry

        lax.fori_loop(0, (nchunk - 5) // 3, body, 0)

        for c in (nchunk - 4, nchunk - 3):
            wait_gather(c, c % 3)
            start_write(c, c % 3)
            wait_write(c - 1, (c - 1) % 3)
            start_gather(c + 2, (c + 2) % 3)
        for c in (nchunk - 2, nchunk - 1):
            wait_gather(c, c % 3)
            start_write(c, c % 3)
            wait_write(c - 1, (c - 1) % 3)
        wait_write(nchunk - 1, (nchunk - 1) % 3)

    return gather_kernel


def kernel(input_ids, word_embeddings):
    b, s = input_ids.shape
    perm_idx = input_ids.T.reshape(-1).astype(jnp.int32)
    return _build(b * s)(perm_idx, word_embeddings)
